# baseline (speedup 1.0000x reference)
import functools

import jax
import jax.numpy as jnp
from jax import lax
from jax.experimental import pallas as pl
from jax.experimental.pallas import tpu as pltpu
from jax.experimental.pallas import tpu_sc as plsc

_B = 4
_N = 8192
_M = 1024
_S = 16
_R2 = 1.0
_TM = 128
_D = 16
_NW = 32
_BPW = (_B * _M * _S) // _NW


_H = _N // 2


def _fps_body(x_ref, y_ref, z_ref, idx_ref, qx_ref, qy_ref, qz_ref):
    X = x_ref[...]
    Y = y_ref[...]
    Z = z_ref[...]
    S3 = jnp.concatenate([X, Y, Z], axis=0)
    half = lax.broadcasted_iota(jnp.int32, (2 * _B, _H), 0) & 1
    gl = lax.broadcasted_iota(jnp.int32, (2 * _B, _H), 1) + _H * half
    gl3 = jnp.concatenate([gl, gl, gl], axis=0)
    _MH = _M // 2
    colh = (lax.broadcasted_iota(jnp.int32, (2 * _B, _MH), 1)
            + _MH * (lax.broadcasted_iota(jnp.int32, (2 * _B, _MH), 0) & 1))

    def withpair(e, op):
        odd = (lax.broadcasted_iota(jnp.int32, e.shape, 0) & 1) == 1
        partner = jnp.where(odd, pltpu.roll(e, 1, 0),
                            pltpu.roll(e, e.shape[0] - 1, 0))
        return op(e, partner)

    def extract(last8):
        l3 = jnp.concatenate([last8, last8, last8], axis=0)
        ext = jnp.sum(jnp.where(gl3 == l3, S3, 0.0), axis=1, keepdims=True)
        s = withpair(ext, jnp.add)
        return s[0:2 * _B], s[2 * _B:4 * _B], s[4 * _B:6 * _B]

    def body(i, st):
        dists, idxs, qx, qy, qz, last8 = st
        px, py, pz = extract(last8)
        qx = jnp.where(colh == (i - 1), px, qx)
        qy = jnp.where(colh == (i - 1), py, qy)
        qz = jnp.where(colh == (i - 1), pz, qz)
        dx = X - px
        dy = Y - py
        dz = Z - pz
        d = dx * dx + dy * dy
        d = d + dz * dz
        dists = jnp.minimum(dists, d)
        mx = withpair(jnp.max(dists, axis=1, keepdims=True), jnp.maximum)
        nxt8 = jnp.min(jnp.where(dists == mx, gl, _N), axis=1, keepdims=True)
        nxt = withpair(nxt8, jnp.minimum)
        idxs = jnp.where(colh == i, nxt, idxs)
        return dists, idxs, qx, qy, qz, nxt

    dists0 = jnp.full((2 * _B, _H), 1e10, jnp.float32)
    idxs0 = jnp.zeros((2 * _B, _MH), jnp.int32)
    q0 = jnp.zeros((2 * _B, _MH), jnp.float32)
    last0 = jnp.zeros((2 * _B, 1), jnp.int32)
    st = lax.fori_loop(1, _M, body, (dists0, idxs0, q0, q0, q0, last0))
    dists, idxs, qx, qy, qz, last8 = st
    px, py, pz = extract(last8)
    qx = jnp.where(colh == (_M - 1), px, qx)
    qy = jnp.where(colh == (_M - 1), py, qy)
    qz = jnp.where(colh == (_M - 1), pz, qz)
    idx_ref[...] = idxs
    qx_ref[...] = qx
    qy_ref[...] = qy
    qz_ref[...] = qz


def _run_fps(X, Y, Z):
    idxs, qx, qy, qz = pl.pallas_call(
        _fps_body,
        out_shape=(
            jax.ShapeDtypeStruct((2 * _B, _M // 2), jnp.int32),
            jax.ShapeDtypeStruct((2 * _B, _M // 2), jnp.float32),
            jax.ShapeDtypeStruct((2 * _B, _M // 2), jnp.float32),
            jax.ShapeDtypeStruct((2 * _B, _M // 2), jnp.float32),
        ),
    )(X.reshape(2 * _B, _H), Y.reshape(2 * _B, _H), Z.reshape(2 * _B, _H))
    return (idxs.reshape(_B, _M), qx.reshape(_B, _M),
            qy.reshape(_B, _M), qz.reshape(_B, _M))


def _ball_body(q_ref, p_ref, o_ref):
    q = q_ref[...].reshape(_TM, 3)
    qx = q[:, 0:1]
    qy = q[:, 1:2]
    qz = q[:, 2:3]
    P = p_ref[...].reshape(3, _N)
    X = P[0:1, :]
    Y = P[1:2, :]
    Z = P[2:3, :]
    xx = qx * qx + qy * qy
    xx = xx + qz * qz
    yy = X * X + Y * Y
    yy = yy + Z * Z
    cross = jnp.dot(q.astype(jnp.bfloat16), P.astype(jnp.bfloat16),
                    preferred_element_type=jnp.float32)
    d2 = (xx + yy) - 2.0 * cross
    d2 = jnp.maximum(d2, 0.0)
    lane = lax.broadcasted_iota(jnp.int32, (_TM, _N), 1)
    cand = jnp.where(d2 < _R2, lane, _N)
    cols = lax.broadcasted_iota(jnp.int32, (_TM, _S), 1)
    sel = jnp.full((_TM, _S), _N, jnp.int32)
    for s in range(_S):
        v = jnp.min(cand, axis=1, keepdims=True)
        sel = jnp.where(cols == s, v, sel)
        cand = jnp.where(cand == v, _N, cand)
    first = sel[:, 0:1]
    first = jnp.where(first >= _N, 0, first)
    ind = jnp.where(sel >= _N, first, sel)
    o_ref[...] = ind.reshape(1, _TM, _S)


def _run_ball(qpts, points):
    return pl.pallas_call(
        _ball_body,
        grid=(_B, _M // _TM),
        in_specs=[
            pl.BlockSpec((1, _TM, 3), lambda b, t: (b, t, 0)),
            pl.BlockSpec((1, 3, _N), lambda b, t: (b, 0, 0)),
        ],
        out_specs=pl.BlockSpec((1, _TM, _S), lambda b, t: (b, t, 0)),
        out_shape=jax.ShapeDtypeStruct((_B, _M, _S), jnp.int32),
    )(qpts, points)


def _run_sc_gather(table, gidx):
    mesh = plsc.VectorSubcoreMesh(core_axis_name="c", subcore_axis_name="s")

    @functools.partial(
        pl.kernel,
        mesh=mesh,
        out_type=jax.ShapeDtypeStruct((_B * _M * _S, _D), jnp.float32),
        scratch_types=[
            pltpu.VMEM((_BPW,), jnp.int32),
            pltpu.VMEM((_BPW, _D), jnp.float32),
            pltpu.SemaphoreType.DMA,
        ],
        compiler_params=pltpu.CompilerParams(use_tc_tiling_on_sc=False),
    )
    def k(table_hbm, idx_hbm, out_hbm, idx_v, rows_v, sem):
        wid = lax.axis_index("s") * 2 + lax.axis_index("c")
        base = wid * _BPW
        pltpu.sync_copy(idx_hbm.at[pl.ds(base, _BPW)], idx_v)
        pltpu.async_copy(table_hbm.at[idx_v], rows_v, sem).wait()
        pltpu.sync_copy(rows_v, out_hbm.at[pl.ds(base, _BPW)])

    return k(table, gidx)


_P = _B * _M * _S
_RT = 8192
_NT = _P // _RT
_INV_P = 1.0 / _P


def _accum_stats(t, y, s_ref, acc_ref):
    ps = jnp.concatenate(
        [jnp.sum(y, axis=0, keepdims=True),
         jnp.sum(y * y, axis=0, keepdims=True)], axis=0)

    @pl.when(t == 0)
    def _():
        acc_ref[...] = ps

    @pl.when(t > 0)
    def _():
        acc_ref[...] += ps

    @pl.when(t == _NT - 1)
    def _():
        s_ref[...] = acc_ref[...]


def _conv0_body(g_ref, q_ref, w_ref, b_ref, y_ref, s_ref, acc_ref):
    t = pl.program_id(0)
    x = g_ref[...] - q_ref[...]
    y = jnp.dot(x, w_ref[...], preferred_element_type=jnp.float32)
    y = y + b_ref[...]
    y_ref[...] = y
    _accum_stats(t, y, s_ref, acc_ref)


def _normconv_body(yin_ref, sin_ref, gm_ref, bt_ref, w_ref, b_ref,
                   y_ref, s_ref, acc_ref):
    t = pl.program_id(0)
    y0 = yin_ref[...]
    mu = sin_ref[0:1, :] * _INV_P
    var = sin_ref[1:2, :] * _INV_P - mu * mu
    x = (y0 - mu) / jnp.sqrt(var + 1e-3)
    x = jnp.maximum(x * gm_ref[...] + bt_ref[...], 0.0)
    y = jnp.dot(x, w_ref[...], preferred_element_type=jnp.float32)
    y = y + b_ref[...]
    y_ref[...] = y
    _accum_stats(t, y, s_ref, acc_ref)


def _normpool_body(yin_ref, sin_ref, gm_ref, bt_ref, o_ref):
    y0 = yin_ref[...]
    mu = sin_ref[0:1, :] * _INV_P
    var = sin_ref[1:2, :] * _INV_P - mu * mu
    x = (y0 - mu) / jnp.sqrt(var + 1e-3)
    x = jnp.maximum(x * gm_ref[...] + bt_ref[...], 0.0)
    o_ref[...] = jnp.max(x.reshape(_RT // _S, _S, 64), axis=1)


def _full(shape):
    return pl.BlockSpec(shape, lambda t: tuple(0 for _ in shape))


def _run_mlp(g, qe, W0p, b0, g0, e0, W1, b1, g1, e1, W2, b2, g2, e2):
    row = lambda c: pl.BlockSpec((_RT, c), lambda t: (t, 0))
    y0, s0 = pl.pallas_call(
        _conv0_body,
        grid=(_NT,),
        in_specs=[row(_D), row(_D), _full((_D, 32)), _full((1, 32))],
        out_specs=(row(32), _full((2, 32))),
        out_shape=(jax.ShapeDtypeStruct((_P, 32), jnp.float32),
                   jax.ShapeDtypeStruct((2, 32), jnp.float32)),
        scratch_shapes=[pltpu.VMEM((2, 32), jnp.float32)],
    )(g, qe, W0p, b0)

    def normconv(yin, sin, gm, bt, w, b, cin, cout):
        return pl.pallas_call(
            _normconv_body,
            grid=(_NT,),
            in_specs=[row(cin), _full((2, cin)), _full((1, cin)),
                      _full((1, cin)), _full((cin, cout)), _full((1, cout))],
            out_specs=(row(cout), _full((2, cout))),
            out_shape=(jax.ShapeDtypeStruct((_P, cout), jnp.float32),
                       jax.ShapeDtypeStruct((2, cout), jnp.float32)),
            scratch_shapes=[pltpu.VMEM((2, cout), jnp.float32)],
        )(yin, sin, gm, bt, w, b)

    y1, s1 = normconv(y0, s0, g0, e0, W1, b1, 32, 32)
    y2, s2 = normconv(y1, s1, g1, e1, W2, b2, 32, 64)

    pooled = pl.pallas_call(
        _normpool_body,
        grid=(_NT,),
        in_specs=[row(64), _full((2, 64)), _full((1, 64)), _full((1, 64))],
        out_specs=pl.BlockSpec((_RT // _S, 64), lambda t: (t, 0)),
        out_shape=jax.ShapeDtypeStruct((_B * _M, 64), jnp.float32),
    )(y2, s2, g2, e2)
    return pooled


def kernel(points, features, W0, b0, gamma0, beta0, W1, b1, gamma1, beta1,
           W2, b2, gamma2, beta2):
    X = points[:, 0, :]
    Y = points[:, 1, :]
    Z = points[:, 2, :]
    fps_idx, qx, qy, qz = _run_fps(X, Y, Z)

    qpts = jnp.stack([qx, qy, qz], axis=-1)
    ind = _run_ball(qpts, points)

    pts_rows = jnp.transpose(points, (0, 2, 1))
    feat_rows = jnp.transpose(features, (0, 2, 1))
    pad = jnp.zeros((_B, _N, _D - 6), jnp.float32)
    table = jnp.concatenate([pts_rows, feat_rows, pad], axis=-1)
    table = table.reshape(_B * _N, _D)
    gidx = (ind + (jnp.arange(_B, dtype=jnp.int32) * _N)[:, None, None])
    gidx = gidx.reshape(_B * _M * _S)

    g = _run_sc_gather(table, gidx)

    qpad = jnp.zeros((_B, _M, _D - 3), jnp.float32)
    qrows = jnp.concatenate([qpts, qpad], axis=-1).reshape(_B * _M, _D)
    qe = jnp.repeat(qrows, _S, axis=0)

    W0p = jnp.zeros((_D, 32), jnp.float32).at[:6, :].set(W0)
    as2d = lambda v: v.reshape(1, -1)
    pooled = _run_mlp(g, qe, W0p, as2d(b0), as2d(gamma0), as2d(beta0),
                      W1, as2d(b1), as2d(gamma1), as2d(beta1),
                      W2, as2d(b2), as2d(gamma2), as2d(beta2))

    new_points = jnp.stack([qx, qy, qz], axis=1)
    new_features = jnp.transpose(pooled.reshape(_B, _M, 64), (0, 2, 1))
    return new_points, new_features


# speedup vs baseline: 1.1256x; 1.1256x over previous
import functools

import jax
import jax.numpy as jnp
from jax import lax
from jax.experimental import pallas as pl
from jax.experimental.pallas import tpu as pltpu
from jax.experimental.pallas import tpu_sc as plsc

_B = 4
_N = 8192
_M = 1024
_S = 16
_R2 = 1.0
_TM = 128
_D = 16
_NW = 32
_BPW = (_B * _M * _S) // _NW


def _fps_body(x_ref, y_ref, z_ref, idx_ref, qx_ref, qy_ref, qz_ref):
    X = x_ref[...]
    Y = y_ref[...]
    Z = z_ref[...]
    S3 = jnp.concatenate([X, Y, Z], axis=0)
    lane = lax.broadcasted_iota(jnp.int32, (_B, _N), 1)
    lane3 = lax.broadcasted_iota(jnp.int32, (3 * _B, _N), 1)
    col = lax.broadcasted_iota(jnp.int32, (_B, _M), 1)

    def extract(last):
        l3 = jnp.concatenate([last, last, last], axis=0)
        ext = jnp.sum(jnp.where(lane3 == l3, S3, 0.0), axis=1, keepdims=True)
        return ext[0:_B], ext[_B:2 * _B], ext[2 * _B:3 * _B]

    def body(i, st):
        dists, idxs, qx, qy, qz, last = st
        px, py, pz = extract(last)
        qx = jnp.where(col == (i - 1), px, qx)
        qy = jnp.where(col == (i - 1), py, qy)
        qz = jnp.where(col == (i - 1), pz, qz)
        dx = X - px
        dy = Y - py
        dz = Z - pz
        d = dx * dx + dy * dy
        d = d + dz * dz
        dists = jnp.minimum(dists, d)
        mx = jnp.max(dists, axis=1, keepdims=True)
        nxt = jnp.min(jnp.where(dists == mx, lane, _N), axis=1, keepdims=True)
        idxs = jnp.where(col == i, nxt, idxs)
        return dists, idxs, qx, qy, qz, nxt

    dists0 = jnp.full((_B, _N), 1e10, jnp.float32)
    idxs0 = jnp.zeros((_B, _M), jnp.int32)
    q0 = jnp.zeros((_B, _M), jnp.float32)
    last0 = jnp.zeros((_B, 1), jnp.int32)
    st = lax.fori_loop(1, _M, body, (dists0, idxs0, q0, q0, q0, last0))
    dists, idxs, qx, qy, qz, last = st
    px, py, pz = extract(last)
    qx = jnp.where(col == (_M - 1), px, qx)
    qy = jnp.where(col == (_M - 1), py, qy)
    qz = jnp.where(col == (_M - 1), pz, qz)
    idx_ref[...] = idxs
    qx_ref[...] = qx
    qy_ref[...] = qy
    qz_ref[...] = qz


def _run_fps(X, Y, Z):
    return pl.pallas_call(
        _fps_body,
        out_shape=(
            jax.ShapeDtypeStruct((_B, _M), jnp.int32),
            jax.ShapeDtypeStruct((_B, _M), jnp.float32),
            jax.ShapeDtypeStruct((_B, _M), jnp.float32),
            jax.ShapeDtypeStruct((_B, _M), jnp.float32),
        ),
    )(X, Y, Z)


def _ball_body(q_ref, p_ref, o_ref):
    q = q_ref[...].reshape(_TM, 3)
    qx = q[:, 0:1]
    qy = q[:, 1:2]
    qz = q[:, 2:3]
    P = p_ref[...].reshape(3, _N)
    X = P[0:1, :]
    Y = P[1:2, :]
    Z = P[2:3, :]
    xx = qx * qx + qy * qy
    xx = xx + qz * qz
    yy = X * X + Y * Y
    yy = yy + Z * Z
    cross = jnp.dot(q.astype(jnp.bfloat16), P.astype(jnp.bfloat16),
                    preferred_element_type=jnp.float32)
    d2 = (xx + yy) - 2.0 * cross
    d2 = jnp.maximum(d2, 0.0)
    lane = lax.broadcasted_iota(jnp.int32, (_TM, _N), 1)
    cand = jnp.where(d2 < _R2, lane, _N)
    cols = lax.broadcasted_iota(jnp.int32, (_TM, _S), 1)
    sel = jnp.full((_TM, _S), _N, jnp.int32)
    v = jnp.min(cand, axis=1, keepdims=True)
    sel = jnp.where(cols == 0, v, sel)
    for s in range(1, _S):
        v = jnp.min(jnp.where(cand > v, cand, _N), axis=1, keepdims=True)
        sel = jnp.where(cols == s, v, sel)
    first = sel[:, 0:1]
    first = jnp.where(first >= _N, 0, first)
    ind = jnp.where(sel >= _N, first, sel)
    o_ref[...] = ind.reshape(1, _TM, _S)


def _run_ball(qpts, points):
    return pl.pallas_call(
        _ball_body,
        grid=(_B, _M // _TM),
        in_specs=[
            pl.BlockSpec((1, _TM, 3), lambda b, t: (b, t, 0)),
            pl.BlockSpec((1, 3, _N), lambda b, t: (b, 0, 0)),
        ],
        out_specs=pl.BlockSpec((1, _TM, _S), lambda b, t: (b, t, 0)),
        out_shape=jax.ShapeDtypeStruct((_B, _M, _S), jnp.int32),
    )(qpts, points)


def _run_sc_gather(table, gidx):
    mesh = plsc.VectorSubcoreMesh(core_axis_name="c", subcore_axis_name="s")

    @functools.partial(
        pl.kernel,
        mesh=mesh,
        out_type=jax.ShapeDtypeStruct((_B * _M * _S, _D), jnp.float32),
        scratch_types=[
            pltpu.VMEM((_BPW,), jnp.int32),
            pltpu.VMEM((_BPW, _D), jnp.float32),
            pltpu.SemaphoreType.DMA,
        ],
        compiler_params=pltpu.CompilerParams(use_tc_tiling_on_sc=False),
    )
    def k(table_hbm, idx_hbm, out_hbm, idx_v, rows_v, sem):
        wid = lax.axis_index("s") * 2 + lax.axis_index("c")
        base = wid * _BPW
        pltpu.sync_copy(idx_hbm.at[pl.ds(base, _BPW)], idx_v)
        pltpu.async_copy(table_hbm.at[idx_v], rows_v, sem).wait()
        pltpu.sync_copy(rows_v, out_hbm.at[pl.ds(base, _BPW)])

    return k(table, gidx)


_P = _B * _M * _S
_RT = 8192
_NT = _P // _RT
_INV_P = 1.0 / _P


def _accum_stats(t, y, s_ref, acc_ref):
    ps = jnp.concatenate(
        [jnp.sum(y, axis=0, keepdims=True),
         jnp.sum(y * y, axis=0, keepdims=True)], axis=0)

    @pl.when(t == 0)
    def _():
        acc_ref[...] = ps

    @pl.when(t > 0)
    def _():
        acc_ref[...] += ps

    @pl.when(t == _NT - 1)
    def _():
        s_ref[...] = acc_ref[...]


def _conv0_body(g_ref, q_ref, w_ref, b_ref, y_ref, s_ref, acc_ref):
    t = pl.program_id(0)
    x = g_ref[...] - q_ref[...]
    y = jnp.dot(x, w_ref[...], preferred_element_type=jnp.float32)
    y = y + b_ref[...]
    y_ref[...] = y
    _accum_stats(t, y, s_ref, acc_ref)


def _normconv_body(yin_ref, sin_ref, gm_ref, bt_ref, w_ref, b_ref,
                   y_ref, s_ref, acc_ref):
    t = pl.program_id(0)
    y0 = yin_ref[...]
    mu = sin_ref[0:1, :] * _INV_P
    var = sin_ref[1:2, :] * _INV_P - mu * mu
    x = (y0 - mu) / jnp.sqrt(var + 1e-3)
    x = jnp.maximum(x * gm_ref[...] + bt_ref[...], 0.0)
    y = jnp.dot(x, w_ref[...], preferred_element_type=jnp.float32)
    y = y + b_ref[...]
    y_ref[...] = y
    _accum_stats(t, y, s_ref, acc_ref)


def _normpool_body(yin_ref, sin_ref, gm_ref, bt_ref, o_ref):
    y0 = yin_ref[...]
    mu = sin_ref[0:1, :] * _INV_P
    var = sin_ref[1:2, :] * _INV_P - mu * mu
    x = (y0 - mu) / jnp.sqrt(var + 1e-3)
    x = jnp.maximum(x * gm_ref[...] + bt_ref[...], 0.0)
    o_ref[...] = jnp.max(x.reshape(_RT // _S, _S, 64), axis=1)


def _full(shape):
    return pl.BlockSpec(shape, lambda t: tuple(0 for _ in shape))


def _run_mlp(g, qe, W0p, b0, g0, e0, W1, b1, g1, e1, W2, b2, g2, e2):
    row = lambda c: pl.BlockSpec((_RT, c), lambda t: (t, 0))
    y0, s0 = pl.pallas_call(
        _conv0_body,
        grid=(_NT,),
        in_specs=[row(_D), row(_D), _full((_D, 32)), _full((1, 32))],
        out_specs=(row(32), _full((2, 32))),
        out_shape=(jax.ShapeDtypeStruct((_P, 32), jnp.float32),
                   jax.ShapeDtypeStruct((2, 32), jnp.float32)),
        scratch_shapes=[pltpu.VMEM((2, 32), jnp.float32)],
    )(g, qe, W0p, b0)

    def normconv(yin, sin, gm, bt, w, b, cin, cout):
        return pl.pallas_call(
            _normconv_body,
            grid=(_NT,),
            in_specs=[row(cin), _full((2, cin)), _full((1, cin)),
                      _full((1, cin)), _full((cin, cout)), _full((1, cout))],
            out_specs=(row(cout), _full((2, cout))),
            out_shape=(jax.ShapeDtypeStruct((_P, cout), jnp.float32),
                       jax.ShapeDtypeStruct((2, cout), jnp.float32)),
            scratch_shapes=[pltpu.VMEM((2, cout), jnp.float32)],
        )(yin, sin, gm, bt, w, b)

    y1, s1 = normconv(y0, s0, g0, e0, W1, b1, 32, 32)
    y2, s2 = normconv(y1, s1, g1, e1, W2, b2, 32, 64)

    pooled = pl.pallas_call(
        _normpool_body,
        grid=(_NT,),
        in_specs=[row(64), _full((2, 64)), _full((1, 64)), _full((1, 64))],
        out_specs=pl.BlockSpec((_RT // _S, 64), lambda t: (t, 0)),
        out_shape=jax.ShapeDtypeStruct((_B * _M, 64), jnp.float32),
    )(y2, s2, g2, e2)
    return pooled


def kernel(points, features, W0, b0, gamma0, beta0, W1, b1, gamma1, beta1,
           W2, b2, gamma2, beta2):
    X = points[:, 0, :]
    Y = points[:, 1, :]
    Z = points[:, 2, :]
    fps_idx, qx, qy, qz = _run_fps(X, Y, Z)

    qpts = jnp.stack([qx, qy, qz], axis=-1)
    ind = _run_ball(qpts, points)

    pts_rows = jnp.transpose(points, (0, 2, 1))
    feat_rows = jnp.transpose(features, (0, 2, 1))
    pad = jnp.zeros((_B, _N, _D - 6), jnp.float32)
    table = jnp.concatenate([pts_rows, feat_rows, pad], axis=-1)
    table = table.reshape(_B * _N, _D)
    gidx = (ind + (jnp.arange(_B, dtype=jnp.int32) * _N)[:, None, None])
    gidx = gidx.reshape(_B * _M * _S)

    g = _run_sc_gather(table, gidx)

    qpad = jnp.zeros((_B, _M, _D - 3), jnp.float32)
    qrows = jnp.concatenate([qpts, qpad], axis=-1).reshape(_B * _M, _D)
    qe = jnp.repeat(qrows, _S, axis=0)

    W0p = jnp.zeros((_D, 32), jnp.float32).at[:6, :].set(W0)
    as2d = lambda v: v.reshape(1, -1)
    pooled = _run_mlp(g, qe, W0p, as2d(b0), as2d(gamma0), as2d(beta0),
                      W1, as2d(b1), as2d(gamma1), as2d(beta1),
                      W2, as2d(b2), as2d(gamma2), as2d(beta2))

    new_points = jnp.stack([qx, qy, qz], axis=1)
    new_features = jnp.transpose(pooled.reshape(_B, _M, 64), (0, 2, 1))
    return new_points, new_features


# speedup vs baseline: 1.1480x; 1.0198x over previous
import functools

import jax
import jax.numpy as jnp
from jax import lax
from jax.experimental import pallas as pl
from jax.experimental.pallas import tpu as pltpu
from jax.experimental.pallas import tpu_sc as plsc

_B = 4
_N = 8192
_M = 1024
_S = 16
_R2 = 1.0
_TM = 128
_D = 16
_NW = 32
_BPW = (_B * _M * _S) // _NW


def _fps_body(x_ref, y_ref, z_ref, idx_ref, qx_ref, qy_ref, qz_ref):
    X = x_ref[...]
    Y = y_ref[...]
    Z = z_ref[...]
    S3 = jnp.concatenate([X, Y, Z], axis=0)
    lane = lax.broadcasted_iota(jnp.int32, (_B, _N), 1)
    lane3 = lax.broadcasted_iota(jnp.int32, (3 * _B, _N), 1)
    col = lax.broadcasted_iota(jnp.int32, (_B, _M), 1)

    def extract(last):
        l3 = jnp.concatenate([last, last, last], axis=0)
        ext = jnp.sum(jnp.where(lane3 == l3, S3, 0.0), axis=1, keepdims=True)
        return ext[0:_B], ext[_B:2 * _B], ext[2 * _B:3 * _B]

    def body(i, st):
        dists, idxs, qx, qy, qz, last = st
        px, py, pz = extract(last)
        qx = jnp.where(col == (i - 1), px, qx)
        qy = jnp.where(col == (i - 1), py, qy)
        qz = jnp.where(col == (i - 1), pz, qz)
        dx = X - px
        dy = Y - py
        dz = Z - pz
        d = dx * dx + dy * dy
        d = d + dz * dz
        dists = jnp.minimum(dists, d)
        mx = jnp.max(dists, axis=1, keepdims=True)
        nxt = jnp.min(jnp.where(dists == mx, lane, _N), axis=1, keepdims=True)
        idxs = jnp.where(col == i, nxt, idxs)
        return dists, idxs, qx, qy, qz, nxt

    dists0 = jnp.full((_B, _N), 1e10, jnp.float32)
    idxs0 = jnp.zeros((_B, _M), jnp.int32)
    q0 = jnp.zeros((_B, _M), jnp.float32)
    last0 = jnp.zeros((_B, 1), jnp.int32)
    st = lax.fori_loop(1, _M, body, (dists0, idxs0, q0, q0, q0, last0))
    dists, idxs, qx, qy, qz, last = st
    px, py, pz = extract(last)
    qx = jnp.where(col == (_M - 1), px, qx)
    qy = jnp.where(col == (_M - 1), py, qy)
    qz = jnp.where(col == (_M - 1), pz, qz)
    idx_ref[...] = idxs
    qx_ref[...] = qx
    qy_ref[...] = qy
    qz_ref[...] = qz


def _run_fps(X, Y, Z):
    return pl.pallas_call(
        _fps_body,
        out_shape=(
            jax.ShapeDtypeStruct((_B, _M), jnp.int32),
            jax.ShapeDtypeStruct((_B, _M), jnp.float32),
            jax.ShapeDtypeStruct((_B, _M), jnp.float32),
            jax.ShapeDtypeStruct((_B, _M), jnp.float32),
        ),
    )(X, Y, Z)


def _ball_body(q_ref, p_ref, o_ref):
    q = q_ref[...].reshape(_TM, 3)
    qx = q[:, 0:1]
    qy = q[:, 1:2]
    qz = q[:, 2:3]
    P = p_ref[...].reshape(3, _N)
    X = P[0:1, :]
    Y = P[1:2, :]
    Z = P[2:3, :]
    xx = qx * qx + qy * qy
    xx = xx + qz * qz
    yy = X * X + Y * Y
    yy = yy + Z * Z
    cross = jnp.dot(q.astype(jnp.bfloat16), P.astype(jnp.bfloat16),
                    preferred_element_type=jnp.float32)
    d2 = (xx + yy) - 2.0 * cross
    d2 = jnp.maximum(d2, 0.0)
    lane = lax.broadcasted_iota(jnp.int32, (_TM, _N), 1)
    cand = jnp.where(d2 < _R2, lane, _N)
    cols = lax.broadcasted_iota(jnp.int32, (_TM, _S), 1)
    sel = jnp.full((_TM, _S), _N, jnp.int32)
    for s in range(_S):
        v = jnp.min(cand, axis=1, keepdims=True)
        sel = jnp.where(cols == s, v, sel)
        cand = jnp.where(cand == v, _N, cand)
    first = sel[:, 0:1]
    first = jnp.where(first >= _N, 0, first)
    ind = jnp.where(sel >= _N, first, sel)
    o_ref[...] = ind.reshape(1, _TM, _S)


def _run_ball(qpts, points):
    return pl.pallas_call(
        _ball_body,
        grid=(_B, _M // _TM),
        in_specs=[
            pl.BlockSpec((1, _TM, 3), lambda b, t: (b, t, 0)),
            pl.BlockSpec((1, 3, _N), lambda b, t: (b, 0, 0)),
        ],
        out_specs=pl.BlockSpec((1, _TM, _S), lambda b, t: (b, t, 0)),
        out_shape=jax.ShapeDtypeStruct((_B, _M, _S), jnp.int32),
    )(qpts, points)


def _run_sc_gather(table, gidx):
    mesh = plsc.VectorSubcoreMesh(core_axis_name="c", subcore_axis_name="s")

    @functools.partial(
        pl.kernel,
        mesh=mesh,
        out_type=jax.ShapeDtypeStruct((_B * _M * _S, _D), jnp.float32),
        scratch_types=[
            pltpu.VMEM((_BPW,), jnp.int32),
            pltpu.VMEM((_BPW, _D), jnp.float32),
            pltpu.SemaphoreType.DMA,
        ],
        compiler_params=pltpu.CompilerParams(use_tc_tiling_on_sc=False),
    )
    def k(table_hbm, idx_hbm, out_hbm, idx_v, rows_v, sem):
        wid = lax.axis_index("s") * 2 + lax.axis_index("c")
        base = wid * _BPW
        pltpu.sync_copy(idx_hbm.at[pl.ds(base, _BPW)], idx_v)
        pltpu.async_copy(table_hbm.at[idx_v], rows_v, sem).wait()
        pltpu.sync_copy(rows_v, out_hbm.at[pl.ds(base, _BPW)])

    return k(table, gidx)


_P = _B * _M * _S
_RT = 8192
_NT = _P // _RT
_INV_P = 1.0 / _P


def _accum_stats(t, y, s_ref, acc_ref):
    ps = jnp.concatenate(
        [jnp.sum(y, axis=0, keepdims=True),
         jnp.sum(y * y, axis=0, keepdims=True)], axis=0)

    @pl.when(t == 0)
    def _():
        acc_ref[...] = ps

    @pl.when(t > 0)
    def _():
        acc_ref[...] += ps

    @pl.when(t == _NT - 1)
    def _():
        s_ref[...] = acc_ref[...]


def _conv0_body(g_ref, q_ref, w_ref, b_ref, y_ref, s_ref, acc_ref):
    t = pl.program_id(0)
    g = g_ref[...]
    q = q_ref[...]
    x = (g.reshape(_RT // _S, _S, _D) - q[:, None, :]).reshape(_RT, _D)
    y = jnp.dot(x, w_ref[...], preferred_element_type=jnp.float32)
    y = y + b_ref[...]
    y_ref[...] = y
    _accum_stats(t, y, s_ref, acc_ref)


def _normconv_body(yin_ref, sin_ref, gm_ref, bt_ref, w_ref, b_ref,
                   y_ref, s_ref, acc_ref):
    t = pl.program_id(0)
    y0 = yin_ref[...]
    mu = sin_ref[0:1, :] * _INV_P
    var = sin_ref[1:2, :] * _INV_P - mu * mu
    x = (y0 - mu) / jnp.sqrt(var + 1e-3)
    x = jnp.maximum(x * gm_ref[...] + bt_ref[...], 0.0)
    y = jnp.dot(x, w_ref[...], preferred_element_type=jnp.float32)
    y = y + b_ref[...]
    y_ref[...] = y
    _accum_stats(t, y, s_ref, acc_ref)


def _normpool_body(yin_ref, sin_ref, gm_ref, bt_ref, o_ref):
    y0 = yin_ref[...]
    mu = sin_ref[0:1, :] * _INV_P
    var = sin_ref[1:2, :] * _INV_P - mu * mu
    x = (y0 - mu) / jnp.sqrt(var + 1e-3)
    x = jnp.maximum(x * gm_ref[...] + bt_ref[...], 0.0)
    o_ref[...] = jnp.max(x.reshape(_RT // _S, _S, 64), axis=1)


def _full(shape):
    return pl.BlockSpec(shape, lambda t: tuple(0 for _ in shape))


def _run_mlp(g, qe, W0p, b0, g0, e0, W1, b1, g1, e1, W2, b2, g2, e2):
    row = lambda c: pl.BlockSpec((_RT, c), lambda t: (t, 0))
    y0, s0 = pl.pallas_call(
        _conv0_body,
        grid=(_NT,),
        in_specs=[row(_D), pl.BlockSpec((_RT // _S, _D), lambda t: (t, 0)),
                  _full((_D, 32)), _full((1, 32))],
        out_specs=(row(32), _full((2, 32))),
        out_shape=(jax.ShapeDtypeStruct((_P, 32), jnp.float32),
                   jax.ShapeDtypeStruct((2, 32), jnp.float32)),
        scratch_shapes=[pltpu.VMEM((2, 32), jnp.float32)],
    )(g, qe, W0p, b0)

    def normconv(yin, sin, gm, bt, w, b, cin, cout):
        return pl.pallas_call(
            _normconv_body,
            grid=(_NT,),
            in_specs=[row(cin), _full((2, cin)), _full((1, cin)),
                      _full((1, cin)), _full((cin, cout)), _full((1, cout))],
            out_specs=(row(cout), _full((2, cout))),
            out_shape=(jax.ShapeDtypeStruct((_P, cout), jnp.float32),
                       jax.ShapeDtypeStruct((2, cout), jnp.float32)),
            scratch_shapes=[pltpu.VMEM((2, cout), jnp.float32)],
        )(yin, sin, gm, bt, w, b)

    y1, s1 = normconv(y0, s0, g0, e0, W1, b1, 32, 32)
    y2, s2 = normconv(y1, s1, g1, e1, W2, b2, 32, 64)

    pooled = pl.pallas_call(
        _normpool_body,
        grid=(_NT,),
        in_specs=[row(64), _full((2, 64)), _full((1, 64)), _full((1, 64))],
        out_specs=pl.BlockSpec((_RT // _S, 64), lambda t: (t, 0)),
        out_shape=jax.ShapeDtypeStruct((_B * _M, 64), jnp.float32),
    )(y2, s2, g2, e2)
    return pooled


def kernel(points, features, W0, b0, gamma0, beta0, W1, b1, gamma1, beta1,
           W2, b2, gamma2, beta2):
    X = points[:, 0, :]
    Y = points[:, 1, :]
    Z = points[:, 2, :]
    fps_idx, qx, qy, qz = _run_fps(X, Y, Z)

    qpts = jnp.stack([qx, qy, qz], axis=-1)
    ind = _run_ball(qpts, points)

    pts_rows = jnp.transpose(points, (0, 2, 1))
    feat_rows = jnp.transpose(features, (0, 2, 1))
    pad = jnp.zeros((_B, _N, _D - 6), jnp.float32)
    table = jnp.concatenate([pts_rows, feat_rows, pad], axis=-1)
    table = table.reshape(_B * _N, _D)
    gidx = (ind + (jnp.arange(_B, dtype=jnp.int32) * _N)[:, None, None])
    gidx = gidx.reshape(_B * _M * _S)

    g = _run_sc_gather(table, gidx)

    qpad = jnp.zeros((_B, _M, _D - 3), jnp.float32)
    qrows = jnp.concatenate([qpts, qpad], axis=-1).reshape(_B * _M, _D)

    W0p = jnp.zeros((_D, 32), jnp.float32).at[:6, :].set(W0)
    as2d = lambda v: v.reshape(1, -1)
    pooled = _run_mlp(g, qrows, W0p, as2d(b0), as2d(gamma0), as2d(beta0),
                      W1, as2d(b1), as2d(gamma1), as2d(beta1),
                      W2, as2d(b2), as2d(gamma2), as2d(beta2))

    new_points = jnp.stack([qx, qy, qz], axis=1)
    new_features = jnp.transpose(pooled.reshape(_B, _M, 64), (0, 2, 1))
    return new_points, new_features
